# baseline (device time: 21586 ns/iter reference)
import jax
import jax.numpy as jnp
from jax import lax
from jax.experimental import pallas as pl
from jax.experimental.pallas import tpu as pltpu

N_DEV = 4
K = 16
G_STEPS = 4
CHUNK = 1024
W = 128

_IDX_MASK = -4096
_SIGN_FIX = 0x7FFFFFFF
_SENTINEL = -(2**31)


def _pack(x, col_offset):
    b = lax.bitcast_convert_type(x, jnp.int32)
    key = jnp.where(b >= 0, b, b ^ _SIGN_FIX)
    col = lax.broadcasted_iota(jnp.int32, x.shape, 1) + col_offset
    return (key & _IDX_MASK) | col


def _unpack(p):
    key = p & _IDX_MASK
    b = jnp.where(key >= 0, key, key ^ _SIGN_FIX)
    return lax.bitcast_convert_type(b, jnp.float32)


def _topk_rows_t(a, k):
    vals = []
    for _ in range(k):
        v = jnp.max(a, axis=0, keepdims=True)
        vals.append(v)
        a = jnp.where(a == v, _SENTINEL, a)
    return jnp.concatenate(vals, axis=0)


def _chunk_pool(p):
    m, n = p.shape
    g = n // W
    t = [jnp.full((m, W), _SENTINEL, jnp.int32) for _ in range(4)]
    for j in range(g):
        x0 = p[:, j * W:(j + 1) * W]
        n0 = jnp.maximum(t[0], x0)
        x1 = jnp.minimum(t[0], x0)
        n1 = jnp.maximum(t[1], x1)
        x2 = jnp.minimum(t[1], x1)
        n2 = jnp.maximum(t[2], x2)
        x3 = jnp.minimum(t[2], x2)
        n3 = jnp.maximum(t[3], x3)
        t = [n0, n1, n2, n3]
    t = [jnp.transpose(td) for td in t]
    pool = []
    work = t[0]
    ex = None
    for d, cnt in enumerate((16, 8, 4, 4)):
        if d > 0:
            work = jnp.where(ex, t[d], _SENTINEL)
        for _ in range(cnt):
            v = jnp.max(work, axis=0, keepdims=True)
            pool.append(v)
            work = jnp.where(work == v, _SENTINEL, work)
        hit = work == _SENTINEL
        ex = hit if ex is None else ex & hit
    return jnp.concatenate(pool, axis=0)


def kernel(x):
    m, n_per = x.shape

    def body(x_ref, out_ref, pool_ref, comm_ref, send_sems, recv_sems):
        i = pl.program_id(0)
        my_pos = lax.axis_index("i")

        @pl.when(i == 0)
        def _():
            barrier_sem = pltpu.get_barrier_semaphore()
            for o in range(1, N_DEV):
                pl.semaphore_signal(
                    barrier_sem, inc=1,
                    device_id=(lax.rem(my_pos + o, N_DEV),),
                    device_id_type=pl.DeviceIdType.MESH,
                )
            pl.semaphore_wait(barrier_sem, N_DEV - 1)

        pool_ref[i, :, :] = _chunk_pool(_pack(x_ref[...], i * CHUNK))

        @pl.when(i == G_STEPS - 1)
        def _():
            local = jnp.concatenate(
                [pool_ref[s, :, :] for s in range(G_STEPS)], axis=0
            )
            comm_ref[0, :, :] = _topk_rows_t(local, K)

            rdmas = []
            for o in range(1, N_DEV):
                r = pltpu.make_async_remote_copy(
                    src_ref=comm_ref.at[0],
                    dst_ref=comm_ref.at[o],
                    send_sem=send_sems.at[o - 1],
                    recv_sem=recv_sems.at[o - 1],
                    device_id=(lax.rem(my_pos + o, N_DEV),),
                    device_id_type=pl.DeviceIdType.MESH,
                )
                r.start()
                rdmas.append(r)
            for r in rdmas:
                r.wait()

            allc = jnp.concatenate(
                [comm_ref[s, :, :] for s in range(N_DEV)], axis=0
            )
            idx = lax.broadcasted_iota(jnp.int32, allc.shape, 0)
            allc = (allc & _IDX_MASK) | idx
            out_ref[...] = jnp.transpose(_unpack(_topk_rows_t(allc, K)))

    return pl.pallas_call(
        body,
        grid=(G_STEPS,),
        out_shape=jax.ShapeDtypeStruct((m, K), jnp.float32),
        in_specs=[
            pl.BlockSpec((m, CHUNK), lambda i: (0, i)),
        ],
        out_specs=pl.BlockSpec((m, K), lambda i: (0, 0)),
        scratch_shapes=[
            pltpu.VMEM((G_STEPS, 2 * K, m), jnp.int32),
            pltpu.VMEM((N_DEV, K, m), jnp.int32),
            pltpu.SemaphoreType.DMA((N_DEV - 1,)),
            pltpu.SemaphoreType.DMA((N_DEV - 1,)),
        ],
        compiler_params=pltpu.CompilerParams(
            collective_id=0,
            dimension_semantics=("arbitrary",),
        ),
    )(x)


# device time: 16588 ns/iter; 1.3013x vs baseline; 1.3013x over previous
import jax
import jax.numpy as jnp
from jax import lax
from jax.experimental import pallas as pl
from jax.experimental.pallas import tpu as pltpu

N_DEV = 4
K = 16

_IDX_MASK = -4096
_SIGN_FIX = 0x7FFFFFFF
_SENTINEL = -(2**31)


def _pack(x):
    b = lax.bitcast_convert_type(x, jnp.int32)
    key = jnp.where(b >= 0, b, b ^ _SIGN_FIX)
    col = lax.broadcasted_iota(jnp.int32, x.shape, 1)
    return (key & _IDX_MASK) | col


def _unpack(p):
    key = p & _IDX_MASK
    b = jnp.where(key >= 0, key, key ^ _SIGN_FIX)
    return lax.bitcast_convert_type(b, jnp.float32)


def _topk_rows_t(a, k):
    vals = []
    for _ in range(k):
        v = jnp.max(a, axis=0, keepdims=True)
        vals.append(v)
        a = jnp.where(a == v, _SENTINEL, a)
    return jnp.concatenate(vals, axis=0)


def _staged_pool_t(t, counts=(16, 8, 4, 4)):
    pool = []
    work = t[0]
    ex = None
    for d, cnt in enumerate(counts):
        if d > 0:
            work = jnp.where(ex, t[d], _SENTINEL)
        for _ in range(cnt):
            v = jnp.max(work, axis=0, keepdims=True)
            pool.append(v)
            work = jnp.where(work == v, _SENTINEL, work)
        hit = work == _SENTINEL
        ex = hit if ex is None else ex & hit
    return jnp.concatenate(pool, axis=0)


def _local_candidates(p):
    m, n = p.shape
    g = 32
    w = n // g
    t = [jnp.full((m, w), _SENTINEL, jnp.int32) for _ in range(4)]
    for j in range(g):
        x0 = p[:, j * w:(j + 1) * w]
        n0 = jnp.maximum(t[0], x0)
        x1 = jnp.minimum(t[0], x0)
        n1 = jnp.maximum(t[1], x1)
        x2 = jnp.minimum(t[1], x1)
        n2 = jnp.maximum(t[2], x2)
        x3 = jnp.minimum(t[2], x2)
        n3 = jnp.maximum(t[3], x3)
        t = [n0, n1, n2, n3]
    return _staged_pool_t([jnp.transpose(td) for td in t])


def kernel(x):
    m, n_per = x.shape

    def body(x_ref, out_ref, comm_ref, send_sems, recv_sems):
        my_pos = lax.axis_index("i")

        barrier_sem = pltpu.get_barrier_semaphore()
        for o in range(1, N_DEV):
            pl.semaphore_signal(
                barrier_sem, inc=1,
                device_id=(lax.rem(my_pos + o, N_DEV),),
                device_id_type=pl.DeviceIdType.MESH,
            )
        pl.semaphore_wait(barrier_sem, N_DEV - 1)

        pool = _local_candidates(_pack(x_ref[...]))
        comm_ref[0, :, :] = _topk_rows_t(pool, K)

        rdmas = []
        for o in range(1, N_DEV):
            r = pltpu.make_async_remote_copy(
                src_ref=comm_ref.at[0],
                dst_ref=comm_ref.at[o],
                send_sem=send_sems.at[o - 1],
                recv_sem=recv_sems.at[o - 1],
                device_id=(lax.rem(my_pos + o, N_DEV),),
                device_id_type=pl.DeviceIdType.MESH,
            )
            r.start()
            rdmas.append(r)
        for r in rdmas:
            r.wait()

        allc = jnp.concatenate(
            [comm_ref[s, :, :] for s in range(N_DEV)], axis=0
        )
        idx = lax.broadcasted_iota(jnp.int32, allc.shape, 0)
        allc = (allc & _IDX_MASK) | idx
        out_ref[...] = jnp.transpose(_unpack(_topk_rows_t(allc, K)))

    return pl.pallas_call(
        body,
        out_shape=jax.ShapeDtypeStruct((m, K), jnp.float32),
        in_specs=[pl.BlockSpec(memory_space=pltpu.VMEM)],
        out_specs=pl.BlockSpec(memory_space=pltpu.VMEM),
        scratch_shapes=[
            pltpu.VMEM((N_DEV, K, m), jnp.int32),
            pltpu.SemaphoreType.DMA((N_DEV - 1,)),
            pltpu.SemaphoreType.DMA((N_DEV - 1,)),
        ],
        compiler_params=pltpu.CompilerParams(collective_id=0),
    )(x)
